# baseline (speedup 1.0000x reference)
import functools

import jax
import jax.numpy as jnp
from jax import lax
from jax.experimental import pallas as pl
from jax.experimental.pallas import tpu as pltpu
from jax.experimental.pallas import tpu_sc as plsc

B, S, D = 16, 4096, 1024
NC, NS, L = 2, 16, 16
DH = D // NC
NJ = DH // L
R = 64
NBUF = 2
NCHUNK = S // R

_mesh = plsc.VectorSubcoreMesh(core_axis_name="c", subcore_axis_name="s")


@functools.partial(
    pl.kernel,
    mesh=_mesh,
    out_type=jax.ShapeDtypeStruct((B, D), jnp.float32),
    scratch_types=[
        pltpu.VMEM((NBUF, R, DH), jnp.float32),
        pltpu.VMEM((DH,), jnp.float32),
        pltpu.SemaphoreType.DMA,
        pltpu.SemaphoreType.DMA,
    ],
)
def _mean_pool(x_hbm, out_hbm, buf, acc, sem0, sem1):
    c = lax.axis_index("c")
    s = lax.axis_index("s")
    b = s
    col0 = c * DH
    sems = [sem0, sem1]

    zero = jnp.zeros((L,), jnp.float32)
    for j in range(NJ):
        acc[pl.ds(j * L, L)] = zero

    def chunk_copy(k, i):
        return pltpu.make_async_copy(
            x_hbm.at[b, pl.ds(k * R, R), pl.ds(col0, DH)], buf.at[i], sems[i]
        )

    for i in range(NBUF):
        chunk_copy(i, i).start()

    def outer(kk, carry):
        for i in range(NBUF):
            ki = kk * NBUF + i
            chunk_copy(ki, i).wait()

            def row(r, carry2):
                for j in range(NJ):
                    plsc.addupdate(
                        acc.at[pl.ds(j * L, L)], buf[i, r, pl.ds(j * L, L)]
                    )
                return carry2

            lax.fori_loop(0, R, row, 0, unroll=2)

            @pl.when(ki + NBUF < NCHUNK)
            def _():
                chunk_copy(ki + NBUF, i).start()

        return carry

    lax.fori_loop(0, NCHUNK // NBUF, outer, 0)

    inv = jnp.float32(1.0 / S)
    for j in range(NJ):
        acc[pl.ds(j * L, L)] = acc[pl.ds(j * L, L)] * inv

    pltpu.sync_copy(acc, out_hbm.at[b, pl.ds(col0, DH)])


def kernel(inputs):
    return _mean_pool(inputs)


# speedup vs baseline: 2.7118x; 2.7118x over previous
import functools

import jax
import jax.numpy as jnp
from jax import lax
from jax.experimental import pallas as pl
from jax.experimental.pallas import tpu as pltpu
from jax.experimental.pallas import tpu_sc as plsc

B, S, D = 16, 4096, 1024
NC, NS, L = 2, 16, 16
DH = D // NC
NJ = DH // L
R = 64
NBUF = 2
NCHUNK = S // R

_mesh = plsc.VectorSubcoreMesh(core_axis_name="c", subcore_axis_name="s")


@functools.partial(
    pl.kernel,
    mesh=_mesh,
    out_type=jax.ShapeDtypeStruct((B, D), jnp.float32),
    scratch_types=[
        pltpu.VMEM((NBUF, R, DH), jnp.float32),
        pltpu.VMEM((DH,), jnp.float32),
        pltpu.SemaphoreType.DMA,
        pltpu.SemaphoreType.DMA,
    ],
)
def _mean_pool(x_hbm, out_hbm, buf, acc, sem0, sem1):
    c = lax.axis_index("c")
    s = lax.axis_index("s")
    b = s
    col0 = c * DH
    sems = [sem0, sem1]

    zero = jnp.zeros((L,), jnp.float32)
    for j in range(NJ):
        acc[pl.ds(j * L, L)] = zero

    def chunk_copy(k, i):
        return pltpu.make_async_copy(
            x_hbm.at[b, pl.ds(k * R, R), pl.ds(col0, DH)], buf.at[i], sems[i]
        )

    for i in range(NBUF):
        chunk_copy(i, i).start()

    def outer(kk, carry):
        for i in range(NBUF):
            ki = kk * NBUF + i
            chunk_copy(ki, i).wait()

            @plsc.parallel_loop(0, R, unroll=4)
            def _row(r):
                for j in range(NJ):
                    plsc.addupdate(
                        acc.at[pl.ds(j * L, L)], buf[i, r, pl.ds(j * L, L)]
                    )

            @pl.when(ki + NBUF < NCHUNK)
            def _():
                chunk_copy(ki + NBUF, i).start()

        return carry

    lax.fori_loop(0, NCHUNK // NBUF, outer, 0)

    inv = jnp.float32(1.0 / S)
    for j in range(NJ):
        acc[pl.ds(j * L, L)] = acc[pl.ds(j * L, L)] * inv

    pltpu.sync_copy(acc, out_hbm.at[b, pl.ds(col0, DH)])


def kernel(inputs):
    return _mean_pool(inputs)


# speedup vs baseline: 3.7270x; 1.3743x over previous
import functools

import jax
import jax.numpy as jnp
from jax import lax
from jax.experimental import pallas as pl
from jax.experimental.pallas import tpu as pltpu
from jax.experimental.pallas import tpu_sc as plsc

B, S, D = 16, 4096, 1024
NC, NS, L = 2, 16, 16
DH = D // NC
NJ = DH // L
R = 64
NBUF = 2
NCHUNK = S // R

_mesh = plsc.VectorSubcoreMesh(core_axis_name="c", subcore_axis_name="s")


@functools.partial(
    pl.kernel,
    mesh=_mesh,
    out_type=jax.ShapeDtypeStruct((B, D), jnp.float32),
    scratch_types=[
        pltpu.VMEM((NBUF, R, DH), jnp.float32),
        pltpu.VMEM((DH,), jnp.float32),
        pltpu.SemaphoreType.DMA,
        pltpu.SemaphoreType.DMA,
    ],
)
def _mean_pool(x_hbm, out_hbm, buf, acc, sem0, sem1):
    c = lax.axis_index("c")
    s = lax.axis_index("s")
    b = s
    col0 = c * DH
    sems = [sem0, sem1]

    zero = jnp.zeros((L,), jnp.float32)
    for j in range(NJ):
        acc[pl.ds(j * L, L)] = zero

    def chunk_copy(k, i):
        return pltpu.make_async_copy(
            x_hbm.at[b, pl.ds(k * R, R), pl.ds(col0, DH)], buf.at[i], sems[i]
        )

    for i in range(NBUF):
        chunk_copy(i, i).start()

    zeros16 = (zero,) * (NJ // 2)

    def outer(kk, carry):
        for i in range(NBUF):
            ki = kk * NBUF + i
            chunk_copy(ki, i).wait()

            for p in range(2):
                cb = p * (NJ // 2) * L

                @plsc.parallel_loop(0, R, unroll=2, carry=zeros16)
                def sums(r, accs, i=i, cb=cb):
                    return tuple(
                        accs[j] + buf[i, r, pl.ds(cb + j * L, L)]
                        for j in range(NJ // 2)
                    )

                for j in range(NJ // 2):
                    plsc.addupdate(acc.at[pl.ds(cb + j * L, L)], sums[j])

            @pl.when(ki + NBUF < NCHUNK)
            def _():
                chunk_copy(ki + NBUF, i).start()

        return carry

    lax.fori_loop(0, NCHUNK // NBUF, outer, 0)

    inv = jnp.float32(1.0 / S)
    for j in range(NJ):
        acc[pl.ds(j * L, L)] = acc[pl.ds(j * L, L)] * inv

    pltpu.sync_copy(acc, out_hbm.at[b, pl.ds(col0, DH)])


def kernel(inputs):
    return _mean_pool(inputs)


# speedup vs baseline: 4.3030x; 1.1546x over previous
import functools

import jax
import jax.numpy as jnp
from jax import lax
from jax.experimental import pallas as pl
from jax.experimental.pallas import tpu as pltpu
from jax.experimental.pallas import tpu_sc as plsc

B, S, D = 16, 4096, 1024
NC, NS, L = 2, 16, 16

S_TC = 2560
S_SC = S - S_TC
ROWS_W = S_SC // NC

R = 32
NBUF = 2
NCHUNK = ROWS_W // R
NJ = D // L
GPP = 16
NPASS = NJ // GPP

TBLK = 512

_mesh = plsc.VectorSubcoreMesh(core_axis_name="c", subcore_axis_name="s")


@functools.partial(
    pl.kernel,
    mesh=_mesh,
    out_type=jax.ShapeDtypeStruct((NC, B, D), jnp.float32),
    scratch_types=[
        pltpu.VMEM((NBUF, R, D), jnp.float32),
        pltpu.VMEM((D,), jnp.float32),
        pltpu.SemaphoreType.DMA,
        pltpu.SemaphoreType.DMA,
    ],
)
def _sc_partial(x_hbm, out_hbm, buf, acc, sem0, sem1):
    c = lax.axis_index("c")
    s = lax.axis_index("s")
    b = s
    row0 = S_TC + c * ROWS_W
    sems = [sem0, sem1]

    zero = jnp.zeros((L,), jnp.float32)
    for j in range(NJ):
        acc[pl.ds(j * L, L)] = zero

    def chunk_copy(k, i):
        return pltpu.make_async_copy(
            x_hbm.at[b, pl.ds(row0 + k * R, R)], buf.at[i], sems[i]
        )

    for i in range(NBUF):
        chunk_copy(i, i).start()

    zeros_pass = (zero,) * GPP

    def outer(kk, carry):
        for i in range(NBUF):
            ki = kk * NBUF + i
            chunk_copy(ki, i).wait()

            for p in range(NPASS):
                cb = p * GPP * L

                @plsc.parallel_loop(0, R, unroll=2, carry=zeros_pass)
                def sums(r, accs, i=i, cb=cb):
                    return tuple(
                        accs[j] + buf[i, r, pl.ds(cb + j * L, L)]
                        for j in range(GPP)
                    )

                for j in range(GPP):
                    plsc.addupdate(acc.at[pl.ds(cb + j * L, L)], sums[j])

            @pl.when(ki + NBUF < NCHUNK)
            def _():
                chunk_copy(ki + NBUF, i).start()

        return carry

    lax.fori_loop(0, NCHUNK // NBUF, outer, 0)

    pltpu.sync_copy(acc, out_hbm.at[c, b])


def _tc_body(x_ref, o_ref):
    @pl.when(pl.program_id(1) == 0)
    def _():
        o_ref[...] = jnp.zeros_like(o_ref)

    o_ref[...] += jnp.sum(x_ref[...], axis=1, keepdims=True)


_tc_partial = pl.pallas_call(
    _tc_body,
    grid=(B, S_TC // TBLK),
    in_specs=[pl.BlockSpec((1, TBLK, D), lambda b, k: (b, k, 0))],
    out_specs=pl.BlockSpec((1, 1, D), lambda b, k: (b, 0, 0)),
    out_shape=jax.ShapeDtypeStruct((B, 1, D), jnp.float32),
    compiler_params=pltpu.CompilerParams(
        dimension_semantics=("parallel", "arbitrary")
    ),
)


def _combine_body(t_ref, s_ref, o_ref):
    o_ref[...] = (t_ref[:, 0, :] + s_ref[0] + s_ref[1]) * (1.0 / S)


_combine = pl.pallas_call(
    _combine_body,
    out_shape=jax.ShapeDtypeStruct((B, D), jnp.float32),
)


def kernel(inputs):
    sc_sum = _sc_partial(inputs)
    tc_sum = _tc_partial(inputs)
    return _combine(tc_sum, sc_sum)


# speedup vs baseline: 5.6007x; 1.3016x over previous
import jax
import jax.numpy as jnp
from jax.experimental import pallas as pl
from jax.experimental.pallas import tpu as pltpu

B, S, D = 16, 4096, 1024
TBLK = 1024


def _tc_body(x_ref, o_ref):
    @pl.when(pl.program_id(1) == 0)
    def _():
        o_ref[...] = jnp.zeros_like(o_ref)

    o_ref[...] += jnp.sum(x_ref[...], axis=1, keepdims=True) * (1.0 / S)


_tc_mean = pl.pallas_call(
    _tc_body,
    grid=(B, S // TBLK),
    in_specs=[pl.BlockSpec((1, TBLK, D), lambda b, k: (b, k, 0))],
    out_specs=pl.BlockSpec((1, 1, D), lambda b, k: (b, 0, 0)),
    out_shape=jax.ShapeDtypeStruct((B, 1, D), jnp.float32),
    compiler_params=pltpu.CompilerParams(
        dimension_semantics=("parallel", "arbitrary")
    ),
)


def kernel(inputs):
    return _tc_mean(inputs)[:, 0, :]
